# baseline (speedup 1.0000x reference)
import functools

import jax
import jax.numpy as jnp
from jax import lax
from jax.experimental import pallas as pl
from jax.experimental.pallas import tpu as pltpu
from jax.experimental.pallas import tpu_sc as plsc

_CH = 128
_B = 4096 * 200
_NC = 2
_NS = 16
_NW = _NC * _NS
_BPW = _B // _NW
_CHUNK = 128
_NSTEP = _BPW // _CHUNK
_NBUF = 6
_NT = _NSTEP // 3


@functools.partial(
    pl.kernel,
    mesh=plsc.VectorSubcoreMesh(core_axis_name="c", subcore_axis_name="s"),
    out_type=jax.ShapeDtypeStruct((_B // _CHUNK, _CHUNK, _CH), jnp.float32),
    scratch_types=[
        pltpu.VMEM((_NSTEP, _CHUNK), jnp.int32),
        pltpu.VMEM((_NBUF, _CHUNK, _CH), jnp.float32),
        pltpu.SemaphoreType.DMA((_NBUF,)),
        pltpu.SemaphoreType.DMA((2,)),
    ],
)
def _pe_gather(pe_hbm, pos_hbm, out_hbm, idx_v, rows_v, gsem, wsem):
    wid = lax.axis_index("s") * _NC + lax.axis_index("c")
    base = wid * _NSTEP
    pltpu.sync_copy(pos_hbm.at[wid], idx_v)

    def _gather(j, b):
        return pltpu.make_async_copy(
            pe_hbm.at[idx_v.at[j]], rows_v.at[b], gsem.at[b]
        )

    def _wb_triple(t, g):
        return pltpu.make_async_copy(
            rows_v.at[pl.ds(3 * g, 3)],
            out_hbm.at[pl.ds(base + 3 * t, 3)],
            wsem.at[g],
        )

    for r in range(8):
        idx_v.at[0][pl.ds(pl.multiple_of(r * 16, 8), 16)] = jnp.full((16,), 12345, jnp.int32)

    def _cgather(b):
        return pltpu.make_async_copy(
            pe_hbm.at[idx_v.at[0]], rows_v.at[b], gsem.at[b]
        )

    for b in range(3):
        _cgather(b).start()

    def triple(t, carry):
        g = t % 2
        for q in range(3):
            _cgather(3 * g + q).wait()
            _cgather(3 * (1 - g) + q).start()
        return carry

    lax.fori_loop(0, _NT, triple, 0)
    for q in range(3):
        _cgather(q).wait()
    pltpu.sync_copy(rows_v.at[pl.ds(0, 3)], out_hbm.at[pl.ds(base, 3)])


def kernel(pe, positions):
    pos = positions.reshape(_NW, _NSTEP, _CHUNK)
    out = _pe_gather(pe, pos)
    return out.reshape(*positions.shape, _CH)


# speedup vs baseline: 97.3350x; 97.3350x over previous
import functools

import jax
import jax.numpy as jnp
from jax import lax
from jax.experimental import pallas as pl
from jax.experimental.pallas import tpu as pltpu
from jax.experimental.pallas import tpu_sc as plsc

_CH = 128
_B = 4096 * 200
_NC = 2
_NS = 16
_NW = _NC * _NS
_BPW = _B // _NW
_CHUNK = 128
_NSTEP = _BPW // _CHUNK
_K = 5
_NG = _NSTEP // _K


@functools.partial(
    pl.kernel,
    mesh=plsc.VectorSubcoreMesh(core_axis_name="c", subcore_axis_name="s"),
    out_type=jax.ShapeDtypeStruct((_B, _CH), jnp.float32),
    scratch_types=[
        pltpu.VMEM((_NSTEP, _CHUNK), jnp.int32),
        pltpu.VMEM((_K, _CHUNK, _CH), jnp.float32),
        pltpu.SemaphoreType.DMA,
        pltpu.SemaphoreType.DMA,
    ],
)
def _pe_gather(pe_hbm, pos_hbm, out_hbm, idx_v, rows_v, gsem, wsem):
    wid = lax.axis_index("s") * _NC + lax.axis_index("c")
    base = wid * _BPW
    pltpu.sync_copy(pos_hbm.at[wid], idx_v)

    def group(g, carry):
        j0 = g * _K
        gc = [
            pltpu.async_copy(pe_hbm.at[idx_v.at[j0 + b]], rows_v.at[b], gsem)
            for b in range(_K)
        ]
        wc = []
        for b in range(_K):
            gc[b].wait()
            wc.append(
                pltpu.async_copy(
                    rows_v.at[b],
                    out_hbm.at[pl.ds(base + (j0 + b) * _CHUNK, _CHUNK)],
                    wsem,
                )
            )
        for b in range(_K):
            wc[b].wait()
        return carry

    lax.fori_loop(0, _NG, group, 0)


def kernel(pe, positions):
    pos = positions.reshape(_NW, _NSTEP, _CHUNK)
    out = _pe_gather(pe, pos)
    return out.reshape(*positions.shape, _CH)
